# baseline (speedup 1.0000x reference)
import dataclasses

import jax
import jax.numpy as jnp
from jax import lax
from jax.experimental import pallas as pl
from jax.experimental.pallas import tpu as pltpu
from jax.experimental.pallas import tpu_sc as plsc

N = 10000
E = 320000
D = 128
CHUNK = 128
NTILES = 32
CPT = 80
NCHUNKS = NTILES * CPT
EP = NCHUNKS * CHUNK
SH_ROWS = 10112
ZPT = SH_ROWS // 16
WPT = 624
DEGW = 128

_MESH = plsc.VectorSubcoreMesh(core_axis_name="c", subcore_axis_name="s")


def _writeout(shared, out, c, s):
    wbase = s * WPT
    pltpu.sync_copy(shared.at[pl.ds(wbase, WPT)],
                    out.at[c].at[pl.ds(wbase, WPT)])

    @pl.when(s == 15)
    def _():
        last = 16 * WPT
        pltpu.sync_copy(shared.at[pl.ds(last, N - last)],
                        out.at[c].at[pl.ds(last, N - last)])


HIST = 10240

_CP_NO_LAYOUT = pltpu.CompilerParams()
if "needs_layout_passes" in pltpu.CompilerParams.__dataclass_fields__:
    _CP_NO_LAYOUT = dataclasses.replace(_CP_NO_LAYOUT,
                                        needs_layout_passes=False)


NPASS = 5
HPASS = CPT // NPASS


def _make_agg(with_hist):
    def body(h_hbm, src_hbm, dst_hbm, z_hbm, *rest):
        if with_hist:
            (agg_out, deg_out, src_v, dst_v, rows_a, rows_b, hist_v,
             sem_a, sem_b, agg_sh) = rest
        else:
            (agg_out, src_v, dst_v, rows_a, rows_b,
             sem_a, sem_b, agg_sh) = rest
        c = lax.axis_index("c")
        s = lax.axis_index("s")
        w = c * 16 + s

        pltpu.sync_copy(z_hbm.at[pl.ds(s * ZPT, ZPT)],
                        agg_sh.at[pl.ds(s * ZPT, ZPT)])
        if with_hist:
            @pl.loop(0, HIST, step=16)
            def _(i):
                hist_v[pl.ds(i, 16)] = jnp.zeros((16,), jnp.float32)
        plsc.subcore_barrier()

        ones16 = jnp.ones((16,), jnp.float32)

        def gather(k, buf, sem):
            pltpu.async_copy(h_hbm.at[src_v.at[k]], buf, sem)

        def wait_gather(k, buf, sem):
            pltpu.make_async_copy(h_hbm.at[src_v.at[k]], buf, sem).wait()

        def hist(j):
            if with_hist:
                @pl.loop(0, CHUNK, step=16)
                def _(k):
                    iv = dst_v[j, pl.ds(k, 16)]
                    plsc.addupdate_scatter(hist_v, [iv], ones16)

        for half in range(NPASS):
            base = w * CPT + half * HPASS
            pltpu.sync_copy(src_hbm.at[pl.ds(base, HPASS)], src_v)
            pltpu.sync_copy(dst_hbm.at[pl.ds(base, HPASS)], dst_v)

            gather(0, rows_a, sem_a)

            @pl.loop(0, HPASS, step=2)
            def _(j):
                wait_gather(j, rows_a, sem_a)
                gather(j + 1, rows_b, sem_b)
                hist(j)
                pltpu.sync_copy(rows_a, agg_sh.at[dst_v.at[j]], add=True)

                @pl.when(j + 2 < HPASS)
                def _():
                    gather(j + 2, rows_a, sem_a)

                wait_gather(j + 1, rows_b, sem_b)
                hist(j + 1)
                pltpu.sync_copy(rows_b, agg_sh.at[dst_v.at[j + 1]],
                                add=True)

        if with_hist:
            pltpu.sync_copy(hist_v, deg_out.at[w])
        plsc.subcore_barrier()
        _writeout(agg_sh, agg_out, c, s)

    out_type = [jax.ShapeDtypeStruct((2, N, D), jnp.float32)]
    scratch = [
        pltpu.VMEM((HPASS, CHUNK), jnp.int32),
        pltpu.VMEM((HPASS, CHUNK), jnp.int32),
        pltpu.VMEM((CHUNK, D), jnp.float32),
        pltpu.VMEM((CHUNK, D), jnp.float32),
    ]
    if with_hist:
        out_type.append(jax.ShapeDtypeStruct((NTILES, HIST), jnp.float32))
        scratch.append(pltpu.VMEM((HIST,), jnp.float32))
    scratch += [
        pltpu.SemaphoreType.DMA,
        pltpu.SemaphoreType.DMA,
        pltpu.VMEM_SHARED((SH_ROWS, D), jnp.float32),
    ]
    kwargs = {}
    if with_hist:
        kwargs["compiler_params"] = _CP_NO_LAYOUT
    return pl.kernel(body, out_type=out_type, mesh=_MESH,
                     scratch_types=scratch, **kwargs)


_sc_agg_hist_kernel = _make_agg(True)
_sc_agg_kernel = _make_agg(False)


def _tc_body(h_ref, a0_ref, a1_ref, d_ref, ws_ref, wn_ref, b_ref, o_ref):
    agg = a0_ref[...] + a1_ref[...]
    deg = jnp.sum(d_ref[...], axis=1, keepdims=True)
    mean = agg / jnp.maximum(deg, 1.0)
    out = jnp.dot(h_ref[...], ws_ref[...], preferred_element_type=jnp.float32,
                  precision=jax.lax.Precision.HIGHEST)
    out = out + jnp.dot(mean, wn_ref[...],
                        preferred_element_type=jnp.float32,
                        precision=jax.lax.Precision.HIGHEST)
    out = out + b_ref[...]
    o_ref[...] = jnp.maximum(out, 0.0)


def _tc_combine(h, agg, dcol, W_self, W_neigh, b):
    R = 1000
    return pl.pallas_call(
        _tc_body,
        grid=(N // R,),
        in_specs=[
            pl.BlockSpec((R, D), lambda i: (i, 0)),
            pl.BlockSpec((R, D), lambda i: (i, 0)),
            pl.BlockSpec((R, D), lambda i: (i, 0)),
            pl.BlockSpec((R, NTILES), lambda i: (i, 0)),
            pl.BlockSpec((D, D), lambda i: (0, 0)),
            pl.BlockSpec((D, D), lambda i: (0, 0)),
            pl.BlockSpec((1, D), lambda i: (0, 0)),
        ],
        out_specs=pl.BlockSpec((R, D), lambda i: (i, 0)),
        out_shape=jax.ShapeDtypeStruct((N, D), jnp.float32),
    )(h, agg[0], agg[1], dcol, W_self, W_neigh, b.reshape(1, D))


def _pad_edges(edge_index):
    npad = EP - E
    pad_src = jnp.arange(npad, dtype=jnp.int32) % N
    pad_dst = N + (jnp.arange(npad, dtype=jnp.int32) % 16)
    src = jnp.concatenate([edge_index[0].astype(jnp.int32), pad_src])
    dst = jnp.concatenate([edge_index[1].astype(jnp.int32), pad_dst])
    return src.reshape(NCHUNKS, CHUNK), dst.reshape(NCHUNKS, CHUNK)


def kernel(x, edge_index, W1_self, W1_neigh, b1, W2_self, W2_neigh, b2):
    src, dst = _pad_edges(edge_index)
    z = jnp.zeros((SH_ROWS, D), jnp.float32)
    agg1, degp = _sc_agg_hist_kernel(x, src, dst, z)
    dcol = jnp.transpose(degp)[:N]
    h1 = _tc_combine(x, agg1, dcol, W1_self, W1_neigh, b1)
    (agg2,) = _sc_agg_kernel(h1, src, dst, z)
    h2 = _tc_combine(h1, agg2, dcol, W2_self, W2_neigh, b2)
    return h2


# speedup vs baseline: 1.0431x; 1.0431x over previous
import dataclasses

import jax
import jax.numpy as jnp
from jax import lax
from jax.experimental import pallas as pl
from jax.experimental.pallas import tpu as pltpu
from jax.experimental.pallas import tpu_sc as plsc

N = 10000
E = 320000
D = 128
CHUNK = 128
NTILES = 32
CPT = 80
NCHUNKS = NTILES * CPT
EP = NCHUNKS * CHUNK
SH_ROWS = 10112
ZPT = SH_ROWS // 16
WPT = 624
DEGW = 128

_MESH = plsc.VectorSubcoreMesh(core_axis_name="c", subcore_axis_name="s")


def _writeout(shared, out, c, s):
    wbase = s * WPT
    pltpu.sync_copy(shared.at[pl.ds(wbase, WPT)],
                    out.at[c].at[pl.ds(wbase, WPT)])

    @pl.when(s == 15)
    def _():
        last = 16 * WPT
        pltpu.sync_copy(shared.at[pl.ds(last, N - last)],
                        out.at[c].at[pl.ds(last, N - last)])


HIST = 10240

_CP_NO_LAYOUT = pltpu.CompilerParams()
if "needs_layout_passes" in pltpu.CompilerParams.__dataclass_fields__:
    _CP_NO_LAYOUT = dataclasses.replace(_CP_NO_LAYOUT,
                                        needs_layout_passes=False)


HPASS = CPT // 2


def _sc_deg_body(dst_hbm, deg_out, dst_v, hist_v):
    c = lax.axis_index("c")
    s = lax.axis_index("s")
    w = c * 16 + s

    @pl.loop(0, HIST, step=16)
    def _(i):
        hist_v[pl.ds(i, 16)] = jnp.zeros((16,), jnp.float32)

    pltpu.sync_copy(dst_hbm.at[pl.ds(w * CPT, CPT)], dst_v)
    ones16 = jnp.ones((16,), jnp.float32)

    @pl.loop(0, CPT)
    def _(j):
        @pl.loop(0, CHUNK, step=16)
        def _(k):
            iv = dst_v[j, pl.ds(k, 16)]
            plsc.addupdate_scatter(hist_v, [iv], ones16)

    pltpu.sync_copy(hist_v, deg_out.at[w])


_sc_deg_kernel = pl.kernel(
    _sc_deg_body,
    out_type=[jax.ShapeDtypeStruct((NTILES, HIST), jnp.float32)],
    mesh=_MESH,
    compiler_params=_CP_NO_LAYOUT,
    scratch_types=[
        pltpu.VMEM((CPT, CHUNK), jnp.int32),
        pltpu.VMEM((HIST,), jnp.float32),
    ],
)


def _sc_agg_body(h_hbm, src_hbm, dst_hbm, z_hbm, agg_out, src_v, dst_v,
                 rows_a, rows_b, sem_a, sem_b, agg_sh):
    c = lax.axis_index("c")
    s = lax.axis_index("s")
    w = c * 16 + s

    pltpu.sync_copy(z_hbm.at[pl.ds(s * ZPT, ZPT)],
                    agg_sh.at[pl.ds(s * ZPT, ZPT)])
    plsc.subcore_barrier()

    def gather(k, buf, sem):
        pltpu.async_copy(h_hbm.at[src_v.at[k]], buf, sem)

    def wait_gather(k, buf, sem):
        pltpu.make_async_copy(h_hbm.at[src_v.at[k]], buf, sem).wait()

    for half in range(2):
        base = w * CPT + half * HPASS
        pltpu.sync_copy(src_hbm.at[pl.ds(base, HPASS)], src_v)
        pltpu.sync_copy(dst_hbm.at[pl.ds(base, HPASS)], dst_v)

        gather(0, rows_a, sem_a)

        @pl.loop(0, HPASS, step=2)
        def _(j):
            wait_gather(j, rows_a, sem_a)
            gather(j + 1, rows_b, sem_b)
            pltpu.sync_copy(rows_a, agg_sh.at[dst_v.at[j]], add=True)

            @pl.when(j + 2 < HPASS)
            def _():
                gather(j + 2, rows_a, sem_a)

            wait_gather(j + 1, rows_b, sem_b)
            pltpu.sync_copy(rows_b, agg_sh.at[dst_v.at[j + 1]], add=True)

    plsc.subcore_barrier()
    _writeout(agg_sh, agg_out, c, s)


_sc_agg_kernel = pl.kernel(
    _sc_agg_body,
    out_type=[jax.ShapeDtypeStruct((2, N, D), jnp.float32)],
    mesh=_MESH,
    scratch_types=[
        pltpu.VMEM((HPASS, CHUNK), jnp.int32),
        pltpu.VMEM((HPASS, CHUNK), jnp.int32),
        pltpu.VMEM((CHUNK, D), jnp.float32),
        pltpu.VMEM((CHUNK, D), jnp.float32),
        pltpu.SemaphoreType.DMA,
        pltpu.SemaphoreType.DMA,
        pltpu.VMEM_SHARED((SH_ROWS, D), jnp.float32),
    ],
)


def _tc_body(h_ref, a0_ref, a1_ref, d_ref, ws_ref, wn_ref, b_ref, o_ref):
    agg = a0_ref[...] + a1_ref[...]
    deg = jnp.sum(d_ref[...], axis=1, keepdims=True)
    mean = agg / jnp.maximum(deg, 1.0)
    out = jnp.dot(h_ref[...], ws_ref[...], preferred_element_type=jnp.float32,
                  precision=jax.lax.Precision.HIGHEST)
    out = out + jnp.dot(mean, wn_ref[...],
                        preferred_element_type=jnp.float32,
                        precision=jax.lax.Precision.HIGHEST)
    out = out + b_ref[...]
    o_ref[...] = jnp.maximum(out, 0.0)


def _tc_combine(h, agg, dcol, W_self, W_neigh, b):
    R = 1000
    return pl.pallas_call(
        _tc_body,
        grid=(N // R,),
        in_specs=[
            pl.BlockSpec((R, D), lambda i: (i, 0)),
            pl.BlockSpec((R, D), lambda i: (i, 0)),
            pl.BlockSpec((R, D), lambda i: (i, 0)),
            pl.BlockSpec((R, NTILES), lambda i: (i, 0)),
            pl.BlockSpec((D, D), lambda i: (0, 0)),
            pl.BlockSpec((D, D), lambda i: (0, 0)),
            pl.BlockSpec((1, D), lambda i: (0, 0)),
        ],
        out_specs=pl.BlockSpec((R, D), lambda i: (i, 0)),
        out_shape=jax.ShapeDtypeStruct((N, D), jnp.float32),
    )(h, agg[0], agg[1], dcol, W_self, W_neigh, b.reshape(1, D))


def _pad_edges(edge_index):
    npad = EP - E
    pad_src = jnp.arange(npad, dtype=jnp.int32) % N
    pad_dst = N + (jnp.arange(npad, dtype=jnp.int32) % 16)
    src = jnp.concatenate([edge_index[0].astype(jnp.int32), pad_src])
    dst = jnp.concatenate([edge_index[1].astype(jnp.int32), pad_dst])
    return src.reshape(NCHUNKS, CHUNK), dst.reshape(NCHUNKS, CHUNK)


def kernel(x, edge_index, W1_self, W1_neigh, b1, W2_self, W2_neigh, b2):
    src, dst = _pad_edges(edge_index)
    z = jnp.zeros((SH_ROWS, D), jnp.float32)
    (degp,) = _sc_deg_kernel(dst)
    dcol = jnp.transpose(degp)[:N]
    (agg1,) = _sc_agg_kernel(x, src, dst, z)
    h1 = _tc_combine(x, agg1, dcol, W1_self, W1_neigh, b1)
    (agg2,) = _sc_agg_kernel(h1, src, dst, z)
    h2 = _tc_combine(h1, agg2, dcol, W2_self, W2_neigh, b2)
    return h2


# speedup vs baseline: 1.0611x; 1.0172x over previous
import dataclasses

import jax
import jax.numpy as jnp
from jax import lax
from jax.experimental import pallas as pl
from jax.experimental.pallas import tpu as pltpu
from jax.experimental.pallas import tpu_sc as plsc

N = 10000
E = 320000
D = 128
CHUNK = 128
NTILES = 32
CPT = 80
NCHUNKS = NTILES * CPT
EP = NCHUNKS * CHUNK
SH_ROWS = 10112
ZPT = SH_ROWS // 16
WPT = 624
DEGW = 128

_MESH = plsc.VectorSubcoreMesh(core_axis_name="c", subcore_axis_name="s")


def _writeout(shared, out, c, s):
    wbase = s * WPT
    pltpu.sync_copy(shared.at[pl.ds(wbase, WPT)],
                    out.at[c].at[pl.ds(wbase, WPT)])

    @pl.when(s == 15)
    def _():
        last = 16 * WPT
        pltpu.sync_copy(shared.at[pl.ds(last, N - last)],
                        out.at[c].at[pl.ds(last, N - last)])


HIST = 10240

_CP_NO_LAYOUT = pltpu.CompilerParams()
if "needs_layout_passes" in pltpu.CompilerParams.__dataclass_fields__:
    _CP_NO_LAYOUT = dataclasses.replace(_CP_NO_LAYOUT,
                                        needs_layout_passes=False)


HPASS = CPT // 2


def _sc_deg_body(dst_hbm, deg_out, dst_v, hist_v):
    c = lax.axis_index("c")
    s = lax.axis_index("s")
    w = c * 16 + s

    @pl.loop(0, HIST, step=16)
    def _(i):
        hist_v[pl.ds(i, 16)] = jnp.zeros((16,), jnp.float32)

    pltpu.sync_copy(dst_hbm.at[pl.ds(w * CPT, CPT)], dst_v)
    ones16 = jnp.ones((16,), jnp.float32)

    @pl.loop(0, CPT)
    def _(j):
        @pl.loop(0, CHUNK, step=16)
        def _(k):
            iv = dst_v[j, pl.ds(k, 16)]
            plsc.addupdate_scatter(hist_v, [iv], ones16)

    pltpu.sync_copy(hist_v, deg_out.at[w])


_sc_deg_kernel = pl.kernel(
    _sc_deg_body,
    out_type=[jax.ShapeDtypeStruct((NTILES, HIST), jnp.float32)],
    mesh=_MESH,
    compiler_params=_CP_NO_LAYOUT,
    scratch_types=[
        pltpu.VMEM((CPT, CHUNK), jnp.int32),
        pltpu.VMEM((HIST,), jnp.float32),
    ],
)


def _sc_agg_body(h_hbm, src_hbm, dst_hbm, agg_out, src_v, dst_v,
                 rows_a, rows_b, sem_a, sem_b, agg_sh):
    c = lax.axis_index("c")
    s = lax.axis_index("s")
    w = c * 16 + s

    @pl.loop(0, CHUNK)
    def _(i):
        @pl.loop(0, D, step=16)
        def _(j):
            rows_a[i, pl.ds(j, 16)] = jnp.zeros((16,), jnp.float32)

    for k in range(4):
        pltpu.sync_copy(rows_a, agg_sh.at[pl.ds(s * ZPT + k * CHUNK,
                                                CHUNK)])
    tail = ZPT - 4 * CHUNK
    pltpu.sync_copy(rows_a.at[pl.ds(0, tail)],
                    agg_sh.at[pl.ds(s * ZPT + 4 * CHUNK, tail)])
    plsc.subcore_barrier()

    def gather(k, buf, sem):
        pltpu.async_copy(h_hbm.at[src_v.at[k]], buf, sem)

    def wait_gather(k, buf, sem):
        pltpu.make_async_copy(h_hbm.at[src_v.at[k]], buf, sem).wait()

    for half in range(2):
        base = w * CPT + half * HPASS
        pltpu.sync_copy(src_hbm.at[pl.ds(base, HPASS)], src_v)
        pltpu.sync_copy(dst_hbm.at[pl.ds(base, HPASS)], dst_v)

        gather(0, rows_a, sem_a)

        @pl.loop(0, HPASS, step=2)
        def _(j):
            wait_gather(j, rows_a, sem_a)
            gather(j + 1, rows_b, sem_b)
            pltpu.sync_copy(rows_a, agg_sh.at[dst_v.at[j]], add=True)

            @pl.when(j + 2 < HPASS)
            def _():
                gather(j + 2, rows_a, sem_a)

            wait_gather(j + 1, rows_b, sem_b)
            pltpu.sync_copy(rows_b, agg_sh.at[dst_v.at[j + 1]], add=True)

    plsc.subcore_barrier()
    _writeout(agg_sh, agg_out, c, s)


_sc_agg_kernel = pl.kernel(
    _sc_agg_body,
    out_type=[jax.ShapeDtypeStruct((2, N, D), jnp.float32)],
    mesh=_MESH,
    scratch_types=[
        pltpu.VMEM((HPASS, CHUNK), jnp.int32),
        pltpu.VMEM((HPASS, CHUNK), jnp.int32),
        pltpu.VMEM((CHUNK, D), jnp.float32),
        pltpu.VMEM((CHUNK, D), jnp.float32),
        pltpu.SemaphoreType.DMA,
        pltpu.SemaphoreType.DMA,
        pltpu.VMEM_SHARED((SH_ROWS, D), jnp.float32),
    ],
)


def _tc_body(h_ref, a0_ref, a1_ref, d_ref, ws_ref, wn_ref, b_ref, o_ref):
    agg = a0_ref[...] + a1_ref[...]
    deg = jnp.sum(d_ref[...], axis=1, keepdims=True)
    mean = agg / jnp.maximum(deg, 1.0)
    out = jnp.dot(h_ref[...], ws_ref[...], preferred_element_type=jnp.float32,
                  precision=jax.lax.Precision.HIGHEST)
    out = out + jnp.dot(mean, wn_ref[...],
                        preferred_element_type=jnp.float32,
                        precision=jax.lax.Precision.HIGHEST)
    out = out + b_ref[...]
    o_ref[...] = jnp.maximum(out, 0.0)


def _tc_combine(h, agg, dcol, W_self, W_neigh, b):
    R = 1000
    return pl.pallas_call(
        _tc_body,
        grid=(N // R,),
        in_specs=[
            pl.BlockSpec((R, D), lambda i: (i, 0)),
            pl.BlockSpec((R, D), lambda i: (i, 0)),
            pl.BlockSpec((R, D), lambda i: (i, 0)),
            pl.BlockSpec((R, NTILES), lambda i: (i, 0)),
            pl.BlockSpec((D, D), lambda i: (0, 0)),
            pl.BlockSpec((D, D), lambda i: (0, 0)),
            pl.BlockSpec((1, D), lambda i: (0, 0)),
        ],
        out_specs=pl.BlockSpec((R, D), lambda i: (i, 0)),
        out_shape=jax.ShapeDtypeStruct((N, D), jnp.float32),
    )(h, agg[0], agg[1], dcol, W_self, W_neigh, b.reshape(1, D))


def _pad_edges(edge_index):
    npad = EP - E
    pad_src = jnp.arange(npad, dtype=jnp.int32) % N
    pad_dst = N + (jnp.arange(npad, dtype=jnp.int32) % 16)
    src = jnp.concatenate([edge_index[0].astype(jnp.int32), pad_src])
    dst = jnp.concatenate([edge_index[1].astype(jnp.int32), pad_dst])
    return src.reshape(NCHUNKS, CHUNK), dst.reshape(NCHUNKS, CHUNK)


def kernel(x, edge_index, W1_self, W1_neigh, b1, W2_self, W2_neigh, b2):
    src, dst = _pad_edges(edge_index)
    (degp,) = _sc_deg_kernel(dst)
    dcol = jnp.transpose(degp)[:N]
    (agg1,) = _sc_agg_kernel(x, src, dst)
    h1 = _tc_combine(x, agg1, dcol, W1_self, W1_neigh, b1)
    (agg2,) = _sc_agg_kernel(h1, src, dst)
    h2 = _tc_combine(h1, agg2, dcol, W2_self, W2_neigh, b2)
    return h2
